# initial kernel scaffold (unmeasured)
import jax
import jax.numpy as jnp
from jax import lax
from jax.experimental import pallas as pl
from jax.experimental.pallas import tpu as pltpu

M = 4096
D = 4096
HALF = M // 2
C = 256
N_CHUNK = HALF // C


def kernel(partial, resid, gamma):
    partial2 = partial.reshape(M, D)
    gamma2 = gamma.reshape(1, D)

    def body(partial_ref, resid_ref, gamma_ref, out_ref,
             recv_buf, vown, vrecv, vresid, vout,
             sem_s1, sem_r1, sem_s2, sem_r2, copy_sems):
        my_x = lax.axis_index("x")
        my_y = lax.axis_index("y")
        nbr_x = (1 - my_x, my_y)
        nbr_y = (my_x, 1 - my_y)

        barrier = pltpu.get_barrier_semaphore()
        for nbr in (nbr_x, nbr_y):
            pl.semaphore_signal(barrier, inc=1, device_id=nbr,
                                device_id_type=pl.DeviceIdType.MESH)
        pl.semaphore_wait(barrier, 2)

        row0 = my_y * HALF

        rdma1 = pltpu.make_async_remote_copy(
            src_ref=partial_ref.at[pl.ds(row0, HALF), :],
            dst_ref=recv_buf,
            send_sem=sem_s1, recv_sem=sem_r1,
            device_id=nbr_x, device_id_type=pl.DeviceIdType.MESH,
        )
        rdma1.start()
        rdma1.wait()

        for c in range(N_CHUNK):
            r = row0 + c * C
            cp0 = pltpu.make_async_copy(
                partial_ref.at[pl.ds(r, C), :], vown, copy_sems.at[0])
            cp1 = pltpu.make_async_copy(
                recv_buf.at[pl.ds(c * C, C), :], vrecv, copy_sems.at[1])
            cp2 = pltpu.make_async_copy(
                resid_ref.at[pl.ds(r, C), :], vresid, copy_sems.at[2])
            cp0.start()
            cp1.start()
            cp2.start()
            cp0.wait()
            cp1.wait()
            cp2.wait()
            y = vown[...] + vrecv[...] + vresid[...]
            rms = jnp.sqrt(jnp.mean(y * y, axis=-1, keepdims=True) + 1e-6)
            vout[...] = y / rms * gamma_ref[...]
            cp3 = pltpu.make_async_copy(
                vout, out_ref.at[pl.ds(r, C), :], copy_sems.at[3])
            cp3.start()
            cp3.wait()

        rdma2 = pltpu.make_async_remote_copy(
            src_ref=out_ref.at[pl.ds(row0, HALF), :],
            dst_ref=out_ref.at[pl.ds(row0, HALF), :],
            send_sem=sem_s2, recv_sem=sem_r2,
            device_id=nbr_y, device_id_type=pl.DeviceIdType.MESH,
        )
        rdma2.start()
        rdma2.wait()

    return pl.pallas_call(
        body,
        out_shape=jax.ShapeDtypeStruct((M, D), jnp.float32),
        in_specs=[
            pl.BlockSpec(memory_space=pl.ANY),
            pl.BlockSpec(memory_space=pl.ANY),
            pl.BlockSpec(memory_space=pltpu.MemorySpace.VMEM),
        ],
        out_specs=pl.BlockSpec(memory_space=pl.ANY),
        scratch_shapes=[
            pl.ANY((HALF, D), jnp.float32),
            pltpu.VMEM((C, D), jnp.float32),
            pltpu.VMEM((C, D), jnp.float32),
            pltpu.VMEM((C, D), jnp.float32),
            pltpu.VMEM((C, D), jnp.float32),
            pltpu.SemaphoreType.DMA,
            pltpu.SemaphoreType.DMA,
            pltpu.SemaphoreType.DMA,
            pltpu.SemaphoreType.DMA,
            pltpu.SemaphoreType.DMA((4,)),
        ],
        compiler_params=pltpu.CompilerParams(collective_id=0),
    )(partial2, resid, gamma2)


# baseline (device time: 832397 ns/iter reference)
import jax
import jax.numpy as jnp
from jax import lax
from jax.experimental import pallas as pl
from jax.experimental.pallas import tpu as pltpu

M = 4096
D = 4096
HALF = M // 2
C = 256
N_CHUNK = HALF // C


def kernel(partial, resid, gamma):
    partial2 = partial.reshape(M, D)
    gamma2 = gamma.reshape(1, D)

    def body(partial_ref, resid_ref, gamma_ref, out_ref, recv_buf,
             vown, vrecv, vresid, vout,
             sem_s1, sem_r1, sem_s2, sem_r2, copy_sems):
        my_x = lax.axis_index("x")
        my_y = lax.axis_index("y")
        nbr_x = (1 - my_x, my_y)
        nbr_y = (my_x, 1 - my_y)

        barrier = pltpu.get_barrier_semaphore()
        for nbr in (nbr_x, nbr_y):
            pl.semaphore_signal(barrier, inc=1, device_id=nbr,
                                device_id_type=pl.DeviceIdType.MESH)
        pl.semaphore_wait(barrier, 2)

        row0 = my_y * HALF

        rdma1 = pltpu.make_async_remote_copy(
            src_ref=partial_ref.at[pl.ds(row0, HALF), :],
            dst_ref=recv_buf,
            send_sem=sem_s1, recv_sem=sem_r1,
            device_id=nbr_x, device_id_type=pl.DeviceIdType.MESH,
        )
        rdma1.start()
        rdma1.wait()

        for c in range(N_CHUNK):
            r = row0 + c * C
            cp0 = pltpu.make_async_copy(
                partial_ref.at[pl.ds(r, C), :], vown, copy_sems.at[0])
            cp1 = pltpu.make_async_copy(
                recv_buf.at[pl.ds(c * C, C), :], vrecv, copy_sems.at[1])
            cp2 = pltpu.make_async_copy(
                resid_ref.at[pl.ds(r, C), :], vresid, copy_sems.at[2])
            cp0.start()
            cp1.start()
            cp2.start()
            cp0.wait()
            cp1.wait()
            cp2.wait()
            y = vown[...] + vrecv[...] + vresid[...]
            rms = jnp.sqrt(jnp.mean(y * y, axis=-1, keepdims=True) + 1e-6)
            vout[...] = y / rms * gamma_ref[...]
            cp3 = pltpu.make_async_copy(
                vout, out_ref.at[pl.ds(r, C), :], copy_sems.at[3])
            cp3.start()
            cp3.wait()

        rdma2 = pltpu.make_async_remote_copy(
            src_ref=out_ref.at[pl.ds(row0, HALF), :],
            dst_ref=out_ref.at[pl.ds(row0, HALF), :],
            send_sem=sem_s2, recv_sem=sem_r2,
            device_id=nbr_y, device_id_type=pl.DeviceIdType.MESH,
        )
        rdma2.start()
        rdma2.wait()

    out, _ = pl.pallas_call(
        body,
        out_shape=(
            jax.ShapeDtypeStruct((M, D), jnp.float32),
            jax.ShapeDtypeStruct((HALF, D), jnp.float32),
        ),
        in_specs=[
            pl.BlockSpec(memory_space=pl.ANY),
            pl.BlockSpec(memory_space=pl.ANY),
            pl.BlockSpec(memory_space=pltpu.MemorySpace.VMEM),
        ],
        out_specs=(
            pl.BlockSpec(memory_space=pl.ANY),
            pl.BlockSpec(memory_space=pl.ANY),
        ),
        scratch_shapes=[
            pltpu.VMEM((C, D), jnp.float32),
            pltpu.VMEM((C, D), jnp.float32),
            pltpu.VMEM((C, D), jnp.float32),
            pltpu.VMEM((C, D), jnp.float32),
            pltpu.SemaphoreType.DMA,
            pltpu.SemaphoreType.DMA,
            pltpu.SemaphoreType.DMA,
            pltpu.SemaphoreType.DMA,
            pltpu.SemaphoreType.DMA((4,)),
        ],
        compiler_params=pltpu.CompilerParams(collective_id=0),
    )(partial2, resid, gamma2)
    return out
